# Initial kernel scaffold; baseline (speedup 1.0000x reference)
import jax
import jax.numpy as jnp
from jax.experimental import pallas as pl


def kernel(alpha, beta):
    raise NotImplementedError("write your pallas kernel here")


# speedup vs baseline: 9.8852x; 9.8852x over previous
import numpy as np
import jax
import jax.numpy as jnp
from jax.experimental import pallas as pl
from jax.experimental.pallas import tpu as pltpu

_MAGIC = 0x1BD11BDA
_R1 = (13, 15, 26, 6)
_R2 = (17, 29, 16, 24)
_M32 = 0xFFFFFFFF


def _tf2x32_py(k1, k2, c1, c2):
    ks0, ks1 = k1, k2
    ks2 = k1 ^ k2 ^ _MAGIC
    x0 = (c1 + ks0) & _M32
    x1 = (c2 + ks1) & _M32

    def four(x0, x1, rs):
        for r in rs:
            x0 = (x0 + x1) & _M32
            x1 = ((x1 << r) | (x1 >> (32 - r))) & _M32
            x1 = x0 ^ x1
        return x0, x1

    x0, x1 = four(x0, x1, _R1); x0 = (x0 + ks1) & _M32; x1 = (x1 + ks2 + 1) & _M32
    x0, x1 = four(x0, x1, _R2); x0 = (x0 + ks2) & _M32; x1 = (x1 + ks0 + 2) & _M32
    x0, x1 = four(x0, x1, _R1); x0 = (x0 + ks0) & _M32; x1 = (x1 + ks1 + 3) & _M32
    x0, x1 = four(x0, x1, _R2); x0 = (x0 + ks1) & _M32; x1 = (x1 + ks2 + 4) & _M32
    x0, x1 = four(x0, x1, _R1); x0 = (x0 + ks2) & _M32; x1 = (x1 + ks0 + 5) & _M32
    return x0, x1


_KA1, _KA2 = _tf2x32_py(0, 42, 0, 0)
_KB1, _KB2 = _tf2x32_py(0, 42, 0, 1)


def _tf2x32(k1, k2, c1, c2):
    sl = jax.lax.shift_left
    sr = jax.lax.shift_right_logical
    ks0, ks1 = k1, k2
    ks2 = k1 ^ k2 ^ np.uint32(_MAGIC)
    x0 = c1 + ks0
    x1 = c2 + ks1

    def four(x0, x1, rs):
        for r in rs:
            x0 = x0 + x1
            x1 = sl(x1, np.uint32(r)) | sr(x1, np.uint32(32 - r))
            x1 = x0 ^ x1
        return x0, x1

    x0, x1 = four(x0, x1, _R1); x0 = x0 + ks1; x1 = x1 + (ks2 + np.uint32(1))
    x0, x1 = four(x0, x1, _R2); x0 = x0 + ks2; x1 = x1 + (ks0 + np.uint32(2))
    x0, x1 = four(x0, x1, _R1); x0 = x0 + ks0; x1 = x1 + (ks1 + np.uint32(3))
    x0, x1 = four(x0, x1, _R2); x0 = x0 + ks1; x1 = x1 + (ks2 + np.uint32(4))
    x0, x1 = four(x0, x1, _R1); x0 = x0 + ks2; x1 = x1 + (ks0 + np.uint32(5))
    return x0, x1


def _bits_to_unit(bits):
    fb = jax.lax.shift_right_logical(bits, np.uint32(9)) | np.uint32(0x3F800000)
    return jax.lax.bitcast_convert_type(fb, jnp.float32) - jnp.float32(1.0)


def _uniform01(k1, k2):
    z = jnp.zeros_like(k1)
    b1, b2 = _tf2x32(k1, k2, z, z)
    f = _bits_to_unit(b1 ^ b2)
    return jnp.maximum(jnp.float32(0.0), f)


_ERFINV_LO = (2.81022636e-08, 3.43273939e-07, -3.5233877e-06, -4.39150654e-06,
              0.00021858087, -0.00125372503, -0.00417768164, 0.246640727,
              1.50140941)
_ERFINV_HI = (-0.000200214257, 0.000100950558, 0.00134934322, -0.00367342844,
              0.00573950773, -0.0076224613, 0.00943887047, 1.00167406,
              2.83297682)


def _erf_inv(x):
    w = -jnp.log1p(-x * x)
    lo_w = w - jnp.float32(2.5)
    hi_w = jnp.sqrt(w) - jnp.float32(3.0)
    p_lo = jnp.full_like(x, np.float32(_ERFINV_LO[0]))
    for cc in _ERFINV_LO[1:]:
        p_lo = np.float32(cc) + p_lo * lo_w
    p_hi = jnp.full_like(x, np.float32(_ERFINV_HI[0]))
    for cc in _ERFINV_HI[1:]:
        p_hi = np.float32(cc) + p_hi * hi_w
    p = jnp.where(w < jnp.float32(5.0), p_lo, p_hi)
    return p * x


_NORM_LO = np.float32(np.nextafter(np.float32(-1.0), np.float32(0.0)))
_NORM_SCALE = np.float32(np.float32(1.0) - _NORM_LO)
_SQRT2 = np.float32(np.sqrt(2.0))


def _normal(k1, k2):
    z = jnp.zeros_like(k1)
    b1, b2 = _tf2x32(k1, k2, z, z)
    f = _bits_to_unit(b1 ^ b2)
    u = jnp.maximum(_NORM_LO, f * _NORM_SCALE + _NORM_LO)
    return _SQRT2 * _erf_inv(u)


def _loggamma(gk1, gk2, alpha):
    z = jnp.zeros_like(gk1)
    one_u = z + np.uint32(1)
    two_u = z + np.uint32(2)
    f1 = jnp.float32(1.0)

    a1, a2 = _tf2x32(gk1, gk2, z, z)
    s1, s2 = _tf2x32(gk1, gk2, z, one_u)

    boost = alpha >= f1
    alpha_b = jnp.where(boost, alpha, alpha + f1)
    d = alpha_b - jnp.float32(1.0 / 3.0)
    c = jnp.float32(1.0 / 3.0) / jnp.sqrt(d)

    def reject(X, V, U):
        c1 = U >= f1 - jnp.float32(0.0331) * (X * X)
        c2 = jnp.log(U) >= X * jnp.float32(0.5) + d * ((f1 - V) + jnp.log(V))
        return c1 & c2

    def outer_cond(carry):
        _, _, X, V, U = carry
        return jnp.any(reject(X, V, U))

    def outer_body(carry):
        k1, k2, X, V, U = carry
        m = reject(X, V, U)
        nk1, nk2 = _tf2x32(k1, k2, z, z)
        xk1, xk2 = _tf2x32(k1, k2, z, one_u)
        uk1, uk2 = _tf2x32(k1, k2, z, two_u)

        def inner_cond(ic):
            _, _, _, v = ic
            return jnp.any(v <= 0)

        def inner_body(ic):
            xk1, xk2, x, v = ic
            act = v <= 0
            nxk1, nxk2 = _tf2x32(xk1, xk2, z, z)
            sk1, sk2 = _tf2x32(xk1, xk2, z, one_u)
            xn = _normal(sk1, sk2)
            vn = f1 + xn * c
            x = jnp.where(act, xn, x)
            v = jnp.where(act, vn, v)
            xk1 = jnp.where(act, nxk1, xk1)
            xk2 = jnp.where(act, nxk2, xk2)
            return xk1, xk2, x, v

        _, _, x, v = jax.lax.while_loop(
            inner_cond, inner_body,
            (xk1, xk2, jnp.zeros_like(alpha), -jnp.ones_like(alpha)))
        Xn = x * x
        Vn = (v * v) * v
        Un = _uniform01(uk1, uk2)
        k1 = jnp.where(m, nk1, k1)
        k2 = jnp.where(m, nk2, k2)
        X = jnp.where(m, Xn, X)
        V = jnp.where(m, Vn, V)
        U = jnp.where(m, Un, U)
        return k1, k2, X, V, U

    init = (a1, a2, jnp.zeros_like(alpha), jnp.ones_like(alpha),
            jnp.full_like(alpha, 2.0))
    _, _, _, V, _ = jax.lax.while_loop(outer_cond, outer_body, init)

    u_exp = _uniform01(s1, s2)
    log_samples = jnp.log1p(-u_exp)
    log_boost = jnp.where(boost | (log_samples == 0), jnp.float32(0.0),
                          log_samples * (f1 / alpha))
    return (jnp.log(d) + jnp.log(V)) + log_boost


_LANES = 128
_BLOCK_ROWS = 64


def _ts_kernel(n_total, block_elems, grid_n):
    def body(alpha_ref, beta_ref, scores_ref, action_ref, best_v, best_i):
        g = pl.program_id(0)
        a = alpha_ref[...]
        b = beta_ref[...]
        shape = a.shape

        base = (g * np.int32(block_elems)).astype(jnp.int32)
        row_i = jax.lax.broadcasted_iota(jnp.int32, shape, 0)
        col_i = jax.lax.broadcasted_iota(jnp.int32, shape, 1)
        lin_i = base + row_i * np.int32(_LANES) + col_i
        lin_u = lin_i.astype(jnp.uint32)

        zu = jnp.zeros_like(lin_u)
        ga1, ga2 = _tf2x32(jnp.full(shape, np.uint32(_KA1)),
                           jnp.full(shape, np.uint32(_KA2)), zu, lin_u)
        gb1, gb2 = _tf2x32(jnp.full(shape, np.uint32(_KB1)),
                           jnp.full(shape, np.uint32(_KB2)), zu, lin_u)

        lga = _loggamma(ga1, ga2, a)
        lgb = _loggamma(gb1, gb2, b)
        log_max = jnp.maximum(lga, lgb)
        sa = jnp.exp(lga - log_max)
        sb = jnp.exp(lgb - log_max)
        scores = sa / (sa + sb)
        scores_ref[...] = scores

        valid = lin_i < np.int32(n_total)
        sc = jnp.where(valid, scores, jnp.float32(-1.0))
        blk_max = jnp.max(sc)
        blk_idx = jnp.min(jnp.where(sc == blk_max, lin_i, np.int32(2**31 - 1)))

        @pl.when(g == 0)
        def _init():
            best_v[0] = jnp.float32(-2.0)
            best_i[0] = jnp.int32(0)

        @pl.when(blk_max > best_v[0])
        def _upd():
            best_v[0] = blk_max
            best_i[0] = blk_idx

        @pl.when(g == np.int32(grid_n - 1))
        def _fin():
            action_ref[0] = best_i[0]

    return body


def kernel(alpha, beta):
    n = alpha.shape[0]
    rows = -(-n // _LANES)
    rows_pad = -(-rows // _BLOCK_ROWS) * _BLOCK_ROWS
    total = rows_pad * _LANES
    grid_n = rows_pad // _BLOCK_ROWS
    block_elems = _BLOCK_ROWS * _LANES

    a2d = jnp.concatenate(
        [alpha, jnp.ones((total - n,), jnp.float32)]).reshape(rows_pad, _LANES)
    b2d = jnp.concatenate(
        [beta, jnp.ones((total - n,), jnp.float32)]).reshape(rows_pad, _LANES)

    scores2d, action1 = pl.pallas_call(
        _ts_kernel(n, block_elems, grid_n),
        grid=(grid_n,),
        in_specs=[
            pl.BlockSpec((_BLOCK_ROWS, _LANES), lambda g: (g, 0)),
            pl.BlockSpec((_BLOCK_ROWS, _LANES), lambda g: (g, 0)),
        ],
        out_specs=[
            pl.BlockSpec((_BLOCK_ROWS, _LANES), lambda g: (g, 0)),
            pl.BlockSpec(memory_space=pltpu.SMEM),
        ],
        out_shape=[
            jax.ShapeDtypeStruct((rows_pad, _LANES), jnp.float32),
            jax.ShapeDtypeStruct((1,), jnp.int32),
        ],
        scratch_shapes=[
            pltpu.SMEM((1,), jnp.float32),
            pltpu.SMEM((1,), jnp.int32),
        ],
    )(a2d, b2d)

    scores = scores2d.reshape(-1)[:n]
    action = action1[0]
    return (action, scores)


# speedup vs baseline: 12.1597x; 1.2301x over previous
import numpy as np
import jax
import jax.numpy as jnp
from jax.experimental import pallas as pl
from jax.experimental.pallas import tpu as pltpu

_MAGIC = 0x1BD11BDA
_R1 = (13, 15, 26, 6)
_R2 = (17, 29, 16, 24)
_M32 = 0xFFFFFFFF


def _tf2x32_py(k1, k2, c1, c2):
    ks0, ks1 = k1, k2
    ks2 = k1 ^ k2 ^ _MAGIC
    x0 = (c1 + ks0) & _M32
    x1 = (c2 + ks1) & _M32

    def four(x0, x1, rs):
        for r in rs:
            x0 = (x0 + x1) & _M32
            x1 = ((x1 << r) | (x1 >> (32 - r))) & _M32
            x1 = x0 ^ x1
        return x0, x1

    x0, x1 = four(x0, x1, _R1); x0 = (x0 + ks1) & _M32; x1 = (x1 + ks2 + 1) & _M32
    x0, x1 = four(x0, x1, _R2); x0 = (x0 + ks2) & _M32; x1 = (x1 + ks0 + 2) & _M32
    x0, x1 = four(x0, x1, _R1); x0 = (x0 + ks0) & _M32; x1 = (x1 + ks1 + 3) & _M32
    x0, x1 = four(x0, x1, _R2); x0 = (x0 + ks1) & _M32; x1 = (x1 + ks2 + 4) & _M32
    x0, x1 = four(x0, x1, _R1); x0 = (x0 + ks2) & _M32; x1 = (x1 + ks0 + 5) & _M32
    return x0, x1


_KA1, _KA2 = _tf2x32_py(0, 42, 0, 0)
_KB1, _KB2 = _tf2x32_py(0, 42, 0, 1)


def _tf2x32(k1, k2, c1, c2):
    sl = jax.lax.shift_left
    sr = jax.lax.shift_right_logical
    ks0, ks1 = k1, k2
    ks2 = k1 ^ k2 ^ np.uint32(_MAGIC)
    x0 = c1 + ks0
    x1 = c2 + ks1

    def four(x0, x1, rs):
        for r in rs:
            x0 = x0 + x1
            x1 = sl(x1, np.uint32(r)) | sr(x1, np.uint32(32 - r))
            x1 = x0 ^ x1
        return x0, x1

    x0, x1 = four(x0, x1, _R1); x0 = x0 + ks1; x1 = x1 + (ks2 + np.uint32(1))
    x0, x1 = four(x0, x1, _R2); x0 = x0 + ks2; x1 = x1 + (ks0 + np.uint32(2))
    x0, x1 = four(x0, x1, _R1); x0 = x0 + ks0; x1 = x1 + (ks1 + np.uint32(3))
    x0, x1 = four(x0, x1, _R2); x0 = x0 + ks1; x1 = x1 + (ks2 + np.uint32(4))
    x0, x1 = four(x0, x1, _R1); x0 = x0 + ks2; x1 = x1 + (ks0 + np.uint32(5))
    return x0, x1


def _bits_to_unit(bits):
    fb = jax.lax.shift_right_logical(bits, np.uint32(9)) | np.uint32(0x3F800000)
    return jax.lax.bitcast_convert_type(fb, jnp.float32) - jnp.float32(1.0)


def _uniform01(k1, k2):
    z = jnp.zeros_like(k1)
    b1, b2 = _tf2x32(k1, k2, z, z)
    f = _bits_to_unit(b1 ^ b2)
    return jnp.maximum(jnp.float32(0.0), f)


_ERFINV_LO = (2.81022636e-08, 3.43273939e-07, -3.5233877e-06, -4.39150654e-06,
              0.00021858087, -0.00125372503, -0.00417768164, 0.246640727,
              1.50140941)
_ERFINV_HI = (-0.000200214257, 0.000100950558, 0.00134934322, -0.00367342844,
              0.00573950773, -0.0076224613, 0.00943887047, 1.00167406,
              2.83297682)


def _erf_inv(x):
    w = -jnp.log1p(-x * x)
    lo_w = w - jnp.float32(2.5)
    hi_w = jnp.sqrt(w) - jnp.float32(3.0)
    p_lo = jnp.full_like(x, np.float32(_ERFINV_LO[0]))
    for cc in _ERFINV_LO[1:]:
        p_lo = np.float32(cc) + p_lo * lo_w
    p_hi = jnp.full_like(x, np.float32(_ERFINV_HI[0]))
    for cc in _ERFINV_HI[1:]:
        p_hi = np.float32(cc) + p_hi * hi_w
    p = jnp.where(w < jnp.float32(5.0), p_lo, p_hi)
    return p * x


_NORM_LO = np.float32(np.nextafter(np.float32(-1.0), np.float32(0.0)))
_NORM_SCALE = np.float32(np.float32(1.0) - _NORM_LO)
_SQRT2 = np.float32(np.sqrt(2.0))


def _normal(k1, k2):
    z = jnp.zeros_like(k1)
    b1, b2 = _tf2x32(k1, k2, z, z)
    f = _bits_to_unit(b1 ^ b2)
    u = jnp.maximum(_NORM_LO, f * _NORM_SCALE + _NORM_LO)
    return _SQRT2 * _erf_inv(u)


def _loggamma(gk1, gk2, alpha):
    z = jnp.zeros_like(gk1)
    one_u = z + np.uint32(1)
    two_u = z + np.uint32(2)
    f1 = jnp.float32(1.0)

    a1, a2 = _tf2x32(gk1, gk2, z, z)
    s1, s2 = _tf2x32(gk1, gk2, z, one_u)

    boost = alpha >= f1
    alpha_b = jnp.where(boost, alpha, alpha + f1)
    d = alpha_b - jnp.float32(1.0 / 3.0)
    c = jnp.float32(1.0 / 3.0) / jnp.sqrt(d)

    def reject(X, V, U):
        c1 = U >= f1 - jnp.float32(0.0331) * (X * X)
        c2 = jnp.log(U) >= X * jnp.float32(0.5) + d * ((f1 - V) + jnp.log(V))
        return c1 & c2

    def draw_v(xk1, xk2):
        sk1, sk2 = _tf2x32(xk1, xk2, z, one_u)
        xn = _normal(sk1, sk2)
        return xn, f1 + xn * c

    def inner(xk1, xk2):
        x, v = draw_v(xk1, xk2)

        def inner_cond(ic):
            return jnp.any(ic[3] != 0)

        def inner_body(ic):
            xk1, xk2, x, acti = ic
            act = acti != 0
            nxk1, nxk2 = _tf2x32(xk1, xk2, z, z)
            xk1 = jnp.where(act, nxk1, xk1)
            xk2 = jnp.where(act, nxk2, xk2)
            xn, vn = draw_v(xk1, xk2)
            x = jnp.where(act, xn, x)
            nact = act & (vn <= 0)
            return xk1, xk2, x, nact.astype(jnp.int32)

        _, _, x, _ = jax.lax.while_loop(
            inner_cond, inner_body, (xk1, xk2, x, (v <= 0).astype(jnp.int32)))
        return x

    def one_round(k1, k2):
        xk1, xk2 = _tf2x32(k1, k2, z, one_u)
        uk1, uk2 = _tf2x32(k1, k2, z, two_u)
        x = inner(xk1, xk2)
        Xn = x * x
        Vn = x * c + f1
        Vn = (Vn * Vn) * Vn
        Un = _uniform01(uk1, uk2)
        return Xn, Vn, Un

    X1, V1, U1 = one_round(a1, a2)
    m1 = reject(X1, V1, U1)

    def outer_cond(carry):
        return jnp.any(carry[3] != 0)

    def outer_body(carry):
        k1, k2, V, mi = carry
        m = mi != 0
        nk1, nk2 = _tf2x32(k1, k2, z, z)
        k1 = jnp.where(m, nk1, k1)
        k2 = jnp.where(m, nk2, k2)
        Xn, Vn, Un = one_round(k1, k2)
        V = jnp.where(m, Vn, V)
        nm = m & reject(Xn, Vn, Un)
        return k1, k2, V, nm.astype(jnp.int32)

    _, _, V, _ = jax.lax.while_loop(
        outer_cond, outer_body, (a1, a2, V1, m1.astype(jnp.int32)))

    u_exp = _uniform01(s1, s2)
    log_samples = jnp.log1p(-u_exp)
    log_boost = jnp.where(boost | (log_samples == 0), jnp.float32(0.0),
                          log_samples * (f1 / alpha))
    return (jnp.log(d) + jnp.log(V)) + log_boost


_LANES = 128
_BLOCK_ROWS = 64


def _ts_kernel(n_total, block_elems, grid_n):
    def body(alpha_ref, beta_ref, scores_ref, action_ref, best_v, best_i):
        g = pl.program_id(0)
        a = alpha_ref[...]
        b = beta_ref[...]
        shape = a.shape

        base = (g * np.int32(block_elems)).astype(jnp.int32)
        row_i = jax.lax.broadcasted_iota(jnp.int32, shape, 0)
        col_i = jax.lax.broadcasted_iota(jnp.int32, shape, 1)
        lin_i = base + row_i * np.int32(_LANES) + col_i
        lin_u = lin_i.astype(jnp.uint32)

        zu = jnp.zeros_like(lin_u)
        ga1, ga2 = _tf2x32(jnp.full(shape, np.uint32(_KA1)),
                           jnp.full(shape, np.uint32(_KA2)), zu, lin_u)
        gb1, gb2 = _tf2x32(jnp.full(shape, np.uint32(_KB1)),
                           jnp.full(shape, np.uint32(_KB2)), zu, lin_u)

        lga = _loggamma(ga1, ga2, a)
        lgb = _loggamma(gb1, gb2, b)
        log_max = jnp.maximum(lga, lgb)
        sa = jnp.exp(lga - log_max)
        sb = jnp.exp(lgb - log_max)
        scores = sa / (sa + sb)
        scores_ref[...] = scores

        valid = lin_i < np.int32(n_total)
        sc = jnp.where(valid, scores, jnp.float32(-1.0))
        blk_max = jnp.max(sc)
        blk_idx = jnp.min(jnp.where(sc == blk_max, lin_i, np.int32(2**31 - 1)))

        @pl.when(g == 0)
        def _init():
            best_v[0] = jnp.float32(-2.0)
            best_i[0] = jnp.int32(0)

        @pl.when(blk_max > best_v[0])
        def _upd():
            best_v[0] = blk_max
            best_i[0] = blk_idx

        @pl.when(g == np.int32(grid_n - 1))
        def _fin():
            action_ref[0] = best_i[0]

    return body


def kernel(alpha, beta):
    n = alpha.shape[0]
    rows = -(-n // _LANES)
    rows_pad = -(-rows // _BLOCK_ROWS) * _BLOCK_ROWS
    total = rows_pad * _LANES
    grid_n = rows_pad // _BLOCK_ROWS
    block_elems = _BLOCK_ROWS * _LANES

    a2d = jnp.concatenate(
        [alpha, jnp.ones((total - n,), jnp.float32)]).reshape(rows_pad, _LANES)
    b2d = jnp.concatenate(
        [beta, jnp.ones((total - n,), jnp.float32)]).reshape(rows_pad, _LANES)

    scores2d, action1 = pl.pallas_call(
        _ts_kernel(n, block_elems, grid_n),
        grid=(grid_n,),
        in_specs=[
            pl.BlockSpec((_BLOCK_ROWS, _LANES), lambda g: (g, 0)),
            pl.BlockSpec((_BLOCK_ROWS, _LANES), lambda g: (g, 0)),
        ],
        out_specs=[
            pl.BlockSpec((_BLOCK_ROWS, _LANES), lambda g: (g, 0)),
            pl.BlockSpec(memory_space=pltpu.SMEM),
        ],
        out_shape=[
            jax.ShapeDtypeStruct((rows_pad, _LANES), jnp.float32),
            jax.ShapeDtypeStruct((1,), jnp.int32),
        ],
        scratch_shapes=[
            pltpu.SMEM((1,), jnp.float32),
            pltpu.SMEM((1,), jnp.int32),
        ],
    )(a2d, b2d)

    scores = scores2d.reshape(-1)[:n]
    action = action1[0]
    return (action, scores)
